# Initial kernel scaffold; baseline (speedup 1.0000x reference)
import jax
import jax.numpy as jnp
from jax.experimental import pallas as pl


def kernel(x_random, x_response, y, E_td, E_wae, w_cat, b_cat):
    raise NotImplementedError("write your pallas kernel here")


# speedup vs baseline: 4.2699x; 4.2699x over previous
import functools

import jax
import jax.numpy as jnp
from jax import lax
from jax.experimental import pallas as pl
from jax.experimental.pallas import tpu as pltpu
from jax.experimental.pallas import tpu_sc as plsc

NC = 2
NS = 16
L = 16
BATCH = 4096
HIST = 50
DIM = 50
DIMP = 56
CLASS_NUM = 4

BAGS_PER_WORKER = BATCH // NS
BAGS_PER_GATHER = 2
IDX_COLS = BAGS_PER_GATHER * HIST
IDX_PAD = 104
IDX_ROWS = BAGS_PER_WORKER // BAGS_PER_GATHER
NBUF = 4
GROUPS = IDX_ROWS // NBUF


def _bag_pipeline(s, x_hbm, tab_hbm, out_hbm, idx_v, gbufs, sems,
                  maxacc, outv, redv, resv, shared):
    pltpu.sync_copy(x_hbm.at[s], idx_v)
    for q in range(4):
        maxacc[q, :] = jnp.full((L,), -jnp.inf, jnp.float32)

    def group(g, carry):
        handles = []
        for b in range(NBUF):
            j = g * NBUF + b
            handles.append(
                pltpu.async_copy(tab_hbm.at[idx_v.at[j]], gbufs[b], sems[b]))
        for b in range(NBUF):
            handles[b].wait()
            gb = gbufs[b]
            for bag in range(BAGS_PER_GATHER):
                base = bag * HIST
                a0 = gb[base, pl.ds(0, L)]
                a1 = gb[base, pl.ds(16, L)]
                a2 = gb[base, pl.ds(32, L)]
                a3 = gb[base, pl.ds(34, L)]
                for r in range(1, HIST):
                    row = base + r
                    a0 = a0 + gb[row, pl.ds(0, L)]
                    a1 = a1 + gb[row, pl.ds(16, L)]
                    a2 = a2 + gb[row, pl.ds(32, L)]
                    a3 = a3 + gb[row, pl.ds(34, L)]
                for q, a in enumerate((a0, a1, a2, a3)):
                    lr = jnp.where(a >= 0.0, a, a * 0.01)
                    maxacc[q, :] = jnp.maximum(maxacc[q, :], lr)
        return carry

    lax.fori_loop(0, GROUPS, group, 0)

    outv[pl.ds(48, L)] = jnp.zeros((L,), jnp.float32)
    outv[pl.ds(0, L)] = maxacc[0, :]
    outv[pl.ds(16, L)] = maxacc[1, :]
    outv[pl.ds(32, L)] = maxacc[2, :]
    outv[pl.ds(34, L)] = maxacc[3, :]
    pltpu.sync_copy(outv, shared.at[s])
    plsc.subcore_barrier()

    @pl.when(s == 0)
    def _():
        pltpu.sync_copy(shared, redv)
        m0 = redv[0, pl.ds(0, L)]
        m1 = redv[0, pl.ds(16, L)]
        m2 = redv[0, pl.ds(32, L)]
        m3 = redv[0, pl.ds(48, L)]
        for t in range(1, NS):
            m0 = jnp.maximum(m0, redv[t, pl.ds(0, L)])
            m1 = jnp.maximum(m1, redv[t, pl.ds(16, L)])
            m2 = jnp.maximum(m2, redv[t, pl.ds(32, L)])
            m3 = jnp.maximum(m3, redv[t, pl.ds(48, L)])
        resv[pl.ds(0, L)] = m0
        resv[pl.ds(16, L)] = m1
        resv[pl.ds(32, L)] = m2
        resv[pl.ds(48, L)] = m3
        pltpu.sync_copy(resv, out_hbm)


def _sc_body(xr_hbm, xs_hbm, etd_hbm, ewae_hbm, outr_hbm, outw_hbm,
             idx_v, g0, g1, g2, g3, maxacc, outv, redv, resv, shared,
             sem0, sem1, sem2, sem3):
    c = lax.axis_index("c")
    s = lax.axis_index("s")
    gbufs = (g0, g1, g2, g3)
    sems = (sem0, sem1, sem2, sem3)

    @pl.when(c == 0)
    def _():
        _bag_pipeline(s, xr_hbm, etd_hbm, outr_hbm, idx_v, gbufs, sems,
                      maxacc, outv, redv, resv, shared)

    @pl.when(c == 1)
    def _():
        _bag_pipeline(s, xs_hbm, ewae_hbm, outw_hbm, idx_v, gbufs, sems,
                      maxacc, outv, redv, resv, shared)


def _head_body(pr_ref, pw_ref, w_ref, b_ref, y_ref, pred_ref, loss_ref):
    pr = pr_ref[0:1, 0:DIM]
    pw = pw_ref[0:1, 0:DIM]
    path = jnp.concatenate([pr, pw], axis=1)
    w = w_ref[...]
    logits = jnp.sum(w * path, axis=1, keepdims=True).T + b_ref[...]
    m = jnp.max(logits, axis=1, keepdims=True)
    e = jnp.exp(logits - m)
    p = e / jnp.sum(e, axis=1, keepdims=True)
    pred_ref[...] = p
    is_one = y_ref[...] == 1.0
    ii = lax.broadcasted_iota(jnp.int32, (1, CLASS_NUM), 1).astype(jnp.float32)
    cand = jnp.where(is_one, ii, jnp.float32(CLASS_NUM))
    idx_first = jnp.min(cand)
    label = jnp.where(idx_first < CLASS_NUM, idx_first, 0.0)
    sel = (ii == label).astype(jnp.float32)
    pm = jnp.max(p, axis=1, keepdims=True)
    lse = pm + jnp.log(jnp.sum(jnp.exp(p - pm), axis=1, keepdims=True))
    p_label = jnp.sum(p * sel, axis=1, keepdims=True)
    loss_ref[...] = lse - p_label


@jax.jit
def kernel(x_random, x_response, y, E_td, E_wae, w_cat, b_cat):
    pad = ((0, 0), (0, 0), (0, IDX_PAD - IDX_COLS))
    xr = jnp.pad(x_random.reshape(NS, IDX_ROWS, IDX_COLS), pad)
    xs = jnp.pad(x_response.reshape(NS, IDX_ROWS, IDX_COLS), pad)
    etd = jnp.pad(E_td, ((0, 0), (0, DIMP - DIM)))
    ewae = jnp.pad(E_wae, ((0, 0), (0, DIMP - DIM)))

    sc = pl.kernel(
        _sc_body,
        out_type=[
            jax.ShapeDtypeStruct((4 * L,), jnp.float32),
            jax.ShapeDtypeStruct((4 * L,), jnp.float32),
        ],
        mesh=plsc.VectorSubcoreMesh(core_axis_name="c", subcore_axis_name="s"),
        compiler_params=pltpu.CompilerParams(use_tc_tiling_on_sc=False),
        scratch_types=[
            pltpu.VMEM((IDX_ROWS, IDX_PAD), jnp.int32),
            pltpu.VMEM((IDX_PAD, DIMP), jnp.float32),
            pltpu.VMEM((IDX_PAD, DIMP), jnp.float32),
            pltpu.VMEM((IDX_PAD, DIMP), jnp.float32),
            pltpu.VMEM((IDX_PAD, DIMP), jnp.float32),
            pltpu.VMEM((4, L), jnp.float32),
            pltpu.VMEM((4 * L,), jnp.float32),
            pltpu.VMEM((NS, 4 * L), jnp.float32),
            pltpu.VMEM((4 * L,), jnp.float32),
            pltpu.VMEM_SHARED((NS, 4 * L), jnp.float32),
            pltpu.SemaphoreType.DMA,
            pltpu.SemaphoreType.DMA,
            pltpu.SemaphoreType.DMA,
            pltpu.SemaphoreType.DMA,
        ],
    )
    path_r, path_w = sc(xr, xs, etd, ewae)

    pred2, loss2 = pl.pallas_call(
        _head_body,
        out_shape=[
            jax.ShapeDtypeStruct((1, CLASS_NUM), jnp.float32),
            jax.ShapeDtypeStruct((1, 1), jnp.float32),
        ],
    )(path_r.reshape(1, 4 * L), path_w.reshape(1, 4 * L),
      w_cat, b_cat.reshape(1, CLASS_NUM), y.reshape(1, CLASS_NUM))

    return (pred2[0], loss2[0, 0])
